# baseline (speedup 1.0000x reference)
import functools

import jax
import jax.numpy as jnp
from jax import lax
from jax.experimental import pallas as pl
from jax.experimental.pallas import tpu as pltpu
from jax.experimental.pallas import tpu_sc as plsc

D = 64
_NC, _NS = 2, 16
_NW = _NC * _NS
_CHUNK = 128


def _sc_gather_body(table_hbm, idx_hbm, out_hbm, idx_v, rows_v, sem):
    wid = lax.axis_index("s") * _NC + lax.axis_index("c")
    n_chunks_per_w = idx_hbm.shape[0] // _NW
    for g in range(n_chunks_per_w):
        chunk = wid * n_chunks_per_w + g
        pltpu.sync_copy(idx_hbm.at[chunk], idx_v)
        pltpu.async_copy(table_hbm.at[idx_v], rows_v, sem).wait()
        pltpu.sync_copy(rows_v, out_hbm.at[pl.ds(chunk * _CHUNK, _CHUNK)])


def _tc_body(h_ref, t_ref, mt_ref, o_ref):
    h = h_ref[...]
    t = t_ref[...]
    pieces = [h[:, a:a + 1] * t for a in range(D)]
    o_mat = jnp.concatenate(pieces, axis=1)
    o_ref[...] = jnp.dot(o_mat, mt_ref[...], preferred_element_type=jnp.float32)


def kernel(data, E_record, M):
    K = M.shape[0]
    B = data.shape[1]
    n_rows = 2 * B
    idx = data.reshape(n_rows // _CHUNK, _CHUNK).astype(jnp.int32)

    sc_gather = pl.kernel(
        _sc_gather_body,
        out_type=jax.ShapeDtypeStruct((n_rows, D), jnp.float32),
        mesh=plsc.VectorSubcoreMesh(core_axis_name="c", subcore_axis_name="s"),
        scratch_types=[
            pltpu.VMEM((_CHUNK,), jnp.int32),
            pltpu.VMEM((_CHUNK, D), jnp.float32),
            pltpu.SemaphoreType.DMA,
        ],
        compiler_params=pltpu.CompilerParams(use_tc_tiling_on_sc=False),
    )
    gathered = sc_gather(E_record, idx)

    mt = M.reshape(K, D * D).T

    BT = 256
    n_bt = B // BT
    out = pl.pallas_call(
        _tc_body,
        grid=(n_bt,),
        in_specs=[
            pl.BlockSpec((BT, D), lambda i: (i, 0)),
            pl.BlockSpec((BT, D), lambda i: (i + n_bt, 0)),
            pl.BlockSpec((D * D, K), lambda i: (0, 0)),
        ],
        out_specs=pl.BlockSpec((BT, K), lambda i: (i, 0)),
        out_shape=jax.ShapeDtypeStruct((B, K), jnp.float32),
    )(gathered, gathered, mt)
    return out


# speedup vs baseline: 1.1548x; 1.1548x over previous
import jax
import jax.numpy as jnp
from jax import lax
from jax.experimental import pallas as pl
from jax.experimental.pallas import tpu as pltpu
from jax.experimental.pallas import tpu_sc as plsc

D = 64
_NC, _NS = 2, 16
_NW = _NC * _NS
_CHUNK = 128


def _sc_gather_body(table_hbm, idx_hbm, out_hbm, idx_v, rows_v, sem):
    wid = lax.axis_index("s") * _NC + lax.axis_index("c")
    n_chunks_per_w = idx_hbm.shape[0] // _NW
    for g in range(n_chunks_per_w):
        chunk = wid * n_chunks_per_w + g
        pltpu.sync_copy(idx_hbm.at[chunk], idx_v)
        pltpu.async_copy(table_hbm.at[idx_v], rows_v, sem).wait()
        pltpu.sync_copy(rows_v, out_hbm.at[pl.ds(chunk * _CHUNK, _CHUNK)])


def _tc_body(gh_ref, gt_ref, ph_ref, pt_ref, m2_ref, g_ref, o_ref):
    gh = gh_ref[...]
    gt = gt_ref[...]
    h = jnp.where(ph_ref[...] > 0, gh[:, D:], gh[:, :D]).astype(jnp.bfloat16)
    t = jnp.where(pt_ref[...] > 0, gt[:, D:], gt[:, :D])
    s = jnp.dot(h, m2_ref[...], preferred_element_type=jnp.float32)
    t2 = jnp.concatenate([t, t], axis=1)
    trep = pltpu.repeat(t2, s.shape[1] // (2 * D), axis=1)
    r = (s * trep).astype(jnp.bfloat16)
    o_ref[...] = jnp.dot(r, g_ref[...], preferred_element_type=jnp.float32)


def _tc_call(gathered, parity, M):
    K = M.shape[0]
    B = gathered.shape[0] // 2
    m2 = M.transpose(1, 0, 2).reshape(D, K * D).astype(jnp.bfloat16)
    gsel = jnp.repeat(jnp.eye(K, dtype=jnp.bfloat16), D, axis=0)

    BT = 256
    n_bt = B // BT
    return pl.pallas_call(
        _tc_body,
        grid=(n_bt,),
        in_specs=[
            pl.BlockSpec((BT, 2 * D), lambda i: (i, 0)),
            pl.BlockSpec((BT, 2 * D), lambda i: (i + n_bt, 0)),
            pl.BlockSpec((BT, 1), lambda i: (i, 0)),
            pl.BlockSpec((BT, 1), lambda i: (i + n_bt, 0)),
            pl.BlockSpec((D, K * D), lambda i: (0, 0)),
            pl.BlockSpec((K * D, K), lambda i: (0, 0)),
        ],
        out_specs=pl.BlockSpec((BT, K), lambda i: (i, 0)),
        out_shape=jax.ShapeDtypeStruct((B, K), jnp.float32),
    )(gathered, gathered, parity, parity, m2, gsel)


def kernel(data, E_record, M):
    n_ent = E_record.shape[0]
    B = data.shape[1]
    n_rows = 2 * B
    idx = data.reshape(-1).astype(jnp.int32)
    idx_pair = (idx // 2).reshape(n_rows // _CHUNK, _CHUNK)
    parity = (idx % 2).astype(jnp.float32).reshape(n_rows, 1)
    table2 = E_record.reshape(n_ent // 2, 2 * D)

    sc_gather = pl.kernel(
        _sc_gather_body,
        out_type=jax.ShapeDtypeStruct((n_rows, 2 * D), jnp.float32),
        mesh=plsc.VectorSubcoreMesh(core_axis_name="c", subcore_axis_name="s"),
        scratch_types=[
            pltpu.VMEM((_CHUNK,), jnp.int32),
            pltpu.VMEM((_CHUNK, 2 * D), jnp.float32),
            pltpu.SemaphoreType.DMA,
        ],
        compiler_params=pltpu.CompilerParams(use_tc_tiling_on_sc=False),
    )
    gathered = sc_gather(table2, idx_pair)
    return _tc_call(gathered, parity, M)


# speedup vs baseline: 1.1995x; 1.0387x over previous
import jax
import jax.numpy as jnp
from jax import lax
from jax.experimental import pallas as pl
from jax.experimental.pallas import tpu as pltpu
from jax.experimental.pallas import tpu_sc as plsc

D = 64
_NC, _NS = 2, 16
_NW = _NC * _NS
_CHUNK = 128


def _sc_gather_body(table_hbm, idx_hbm, out_hbm, idx_v, rows_v, sem):
    wid = lax.axis_index("s") * _NC + lax.axis_index("c")
    n_chunks_per_w = idx_hbm.shape[0] // _NW
    for g in range(n_chunks_per_w):
        chunk = wid * n_chunks_per_w + g
        pltpu.sync_copy(idx_hbm.at[chunk], idx_v)
        pltpu.async_copy(table_hbm.at[idx_v], rows_v, sem).wait()
        pltpu.sync_copy(rows_v, out_hbm.at[pl.ds(chunk * _CHUNK, _CHUNK)])


def _tc_body(h_ref, t_ref, m2_ref, g_ref, o_ref):
    h = h_ref[...].astype(jnp.bfloat16)
    t = t_ref[...].astype(jnp.bfloat16)
    s = jnp.dot(h, m2_ref[...],
                preferred_element_type=jnp.float32).astype(jnp.bfloat16)
    t2 = jnp.concatenate([t, t], axis=1)
    trep = pltpu.repeat(t2, s.shape[1] // (2 * D), axis=1)
    o_ref[...] = jnp.dot(s * trep, g_ref[...],
                         preferred_element_type=jnp.float32)


def _tc_call(gathered, M):
    K = M.shape[0]
    B = gathered.shape[0] // 2
    m2 = M.transpose(1, 0, 2).reshape(D, K * D).astype(jnp.bfloat16)
    gsel = jnp.repeat(jnp.eye(K, dtype=jnp.bfloat16), D, axis=0)

    BT = 256
    n_bt = B // BT
    return pl.pallas_call(
        _tc_body,
        grid=(n_bt,),
        in_specs=[
            pl.BlockSpec((BT, D), lambda i: (i, 0)),
            pl.BlockSpec((BT, D), lambda i: (i + n_bt, 0)),
            pl.BlockSpec((D, K * D), lambda i: (0, 0)),
            pl.BlockSpec((K * D, K), lambda i: (0, 0)),
        ],
        out_specs=pl.BlockSpec((BT, K), lambda i: (i, 0)),
        out_shape=jax.ShapeDtypeStruct((B, K), jnp.float32),
    )(gathered, gathered, m2, gsel)


def kernel(data, E_record, M):
    B = data.shape[1]
    n_rows = 2 * B
    idx = data.reshape(n_rows // _CHUNK, _CHUNK).astype(jnp.int32)

    sc_gather = pl.kernel(
        _sc_gather_body,
        out_type=jax.ShapeDtypeStruct((n_rows, D), jnp.float32),
        mesh=plsc.VectorSubcoreMesh(core_axis_name="c", subcore_axis_name="s"),
        scratch_types=[
            pltpu.VMEM((_CHUNK,), jnp.int32),
            pltpu.VMEM((_CHUNK, D), jnp.float32),
            pltpu.SemaphoreType.DMA,
        ],
        compiler_params=pltpu.CompilerParams(use_tc_tiling_on_sc=False),
    )
    gathered = sc_gather(E_record, idx)
    return _tc_call(gathered, M)
